# Initial kernel scaffold; baseline (speedup 1.0000x reference)
import jax
import jax.numpy as jnp
from jax.experimental import pallas as pl


def kernel(hidden_states, gate_w, w_gate, w_up, w_down):
    raise NotImplementedError("write your pallas kernel here")


# speedup vs baseline: 1.9638x; 1.9638x over previous
import functools

import jax
import jax.numpy as jnp
from jax.experimental import pallas as pl
from jax.experimental.pallas import tpu as pltpu

T = 2048
HIDDEN = 1024
FF = 512
E = 8
TOPK = 2


def _router_body(x_ref, gate_ref, combine_ref):
    x = x_ref[...]
    gw = gate_ref[...]
    logits = jax.lax.dot_general(
        x, gw, (((1,), (1,)), ((), ())), preferred_element_type=jnp.float32
    )
    m = jnp.max(logits, axis=1, keepdims=True)
    ex = jnp.exp(logits - m)
    probs = ex / jnp.sum(ex, axis=1, keepdims=True)
    lane = jax.lax.broadcasted_iota(jnp.int32, probs.shape, 1)
    m1 = jnp.max(probs, axis=1, keepdims=True)
    i1 = jnp.min(jnp.where(probs == m1, lane, E), axis=1, keepdims=True)
    oh1 = lane == i1
    probs2 = jnp.where(oh1, -1.0, probs)
    m2 = jnp.max(probs2, axis=1, keepdims=True)
    i2 = jnp.min(jnp.where(probs2 == m2, lane, E), axis=1, keepdims=True)
    oh2 = lane == i2
    s = m1 + m2
    combine_ref[...] = (
        oh1.astype(jnp.float32) * m1 + oh2.astype(jnp.float32) * m2
    ) / s


def _ffn_body(x_ref, combine_ref, wg_ref, wu_ref, wd_ref, out_ref):
    e = pl.program_id(1)
    x = x_ref[...].astype(jnp.bfloat16)
    g = jax.lax.dot_general(
        x, wg_ref[0], (((1,), (0,)), ((), ())), preferred_element_type=jnp.float32
    )
    u = jax.lax.dot_general(
        x, wu_ref[0], (((1,), (0,)), ((), ())), preferred_element_type=jnp.float32
    )
    h = (g * jax.lax.logistic(g) * u).astype(jnp.bfloat16)
    y = jax.lax.dot_general(
        h, wd_ref[0], (((1,), (0,)), ((), ())), preferred_element_type=jnp.float32
    )
    lane = jax.lax.broadcasted_iota(jnp.int32, (1, E), 1)
    sel = (lane == e).astype(jnp.float32)
    c = jnp.sum(combine_ref[...] * sel, axis=1, keepdims=True)
    contrib = y * c

    @pl.when(e == 0)
    def _():
        out_ref[...] = contrib

    @pl.when(e != 0)
    def _():
        out_ref[...] += contrib


@functools.partial(jax.jit, static_argnames=("interpret",))
def kernel(hidden_states, gate_w, w_gate, w_up, w_down, interpret=False):
    old_shape = hidden_states.shape
    x = hidden_states.reshape(-1, old_shape[-1])

    combine = pl.pallas_call(
        _router_body,
        out_shape=jax.ShapeDtypeStruct((T, E), jnp.float32),
        interpret=interpret,
    )(x, gate_w)

    TH = T // 2
    out = pl.pallas_call(
        _ffn_body,
        grid=(2, E),
        in_specs=[
            pl.BlockSpec((TH, HIDDEN), lambda t, e: (t, 0)),
            pl.BlockSpec((TH, E), lambda t, e: (t, 0)),
            pl.BlockSpec((1, HIDDEN, FF), lambda t, e: (e, 0, 0)),
            pl.BlockSpec((1, HIDDEN, FF), lambda t, e: (e, 0, 0)),
            pl.BlockSpec((1, FF, HIDDEN), lambda t, e: (e, 0, 0)),
        ],
        out_specs=pl.BlockSpec((TH, HIDDEN), lambda t, e: (t, 0)),
        out_shape=jax.ShapeDtypeStruct((T, HIDDEN), jnp.float32),
        compiler_params=pltpu.CompilerParams(
            dimension_semantics=("parallel", "arbitrary")
        ),
        interpret=interpret,
    )(x, combine, w_gate, w_up, w_down)

    return out.reshape(old_shape)
